# baseline (speedup 1.0000x reference)
import functools

import jax
import jax.numpy as jnp
from jax import lax
from jax.experimental import pallas as pl
from jax.experimental.pallas import tpu as pltpu
from jax.experimental.pallas import tpu_sc as plsc

N = 10000
E = 320000
D = 128
C = 40

NC = 2
NS = 16
L = 16
D2 = D // NC
EPS = E // NS
K = 128
CHUNKS = -(-EPS // K)
NBUF = 2
CHUNKS = -(-CHUNKS // NBUF) * NBUF
EPP = CHUNKS * K
RPW = 624
TAIL = N - NS * RPW
ZR = 156

_mesh = plsc.VectorSubcoreMesh(core_axis_name="c", subcore_axis_name="s")

_sc_params = pltpu.CompilerParams(
    needs_layout_passes=False, use_tc_tiling_on_sc=False)


@functools.partial(
    pl.kernel,
    out_type=jax.ShapeDtypeStruct((NC * N, D2), jnp.float32),
    mesh=_mesh,
    scratch_types=[
        pltpu.VMEM((CHUNKS, K), jnp.int32),
        pltpu.VMEM((CHUNKS, K), jnp.int32),
        pltpu.VMEM((CHUNKS, K), jnp.float32),
        [pltpu.VMEM((K, D2), jnp.float32)] * NBUF,
        pltpu.VMEM((ZR, D2), jnp.float32),
        pltpu.VMEM_SHARED((N, D2), jnp.float32),
        [pltpu.SemaphoreType.DMA] * NBUF,
        [pltpu.SemaphoreType.DMA] * NBUF,
    ],
    compiler_params=_sc_params,
)
def _sc_aggregate(x_hbm, src_hbm, dst_hbm, val_hbm, out_hbm,
                  srcv, dstv, valv, rows, zbuf, acc, gsem, ssem):
    c = lax.axis_index("c")
    s = lax.axis_index("s")

    zv = jnp.zeros((L,), jnp.float32)

    @pl.loop(0, ZR)
    def _(i):
        for j in range(D2 // L):
            zbuf[i, pl.ds(j * L, L)] = zv

    @pl.loop(0, RPW // ZR)
    def _(t):
        pltpu.sync_copy(zbuf, acc.at[pl.ds(s * RPW + t * ZR, ZR)])

    @pl.when(s == 0)
    def _():
        pltpu.sync_copy(zbuf.at[pl.ds(0, TAIL)], acc.at[pl.ds(NS * RPW, TAIL)])

    pltpu.sync_copy(src_hbm.at[s], srcv)
    pltpu.sync_copy(dst_hbm.at[s], dstv)
    pltpu.sync_copy(val_hbm.at[s], valv)

    coff = jnp.full((L,), c * N, jnp.int32)

    @pl.loop(0, CHUNKS)
    def _(g):
        for t in range(K // L):
            sl = (g, pl.ds(t * L, L))
            srcv[sl] = srcv[sl] + coff

    plsc.subcore_barrier()

    bcast_dnums = lax.GatherDimensionNumbers(
        offset_dims=(), collapsed_slice_dims=(0,), start_index_map=(0,))

    def _scale(buf, g):
        @pl.loop(0, K // L)
        def _(q):
            val16 = valv[g, pl.ds(q * L, L)]
            for l in range(L):
                v = lax.gather(val16, jnp.full((L, 1), l, jnp.int32),
                               bcast_dnums, (1,),
                               mode=lax.GatherScatterMode.PROMISE_IN_BOUNDS)
                e = q * L + l
                for j in range(D2 // L):
                    sl = (e, pl.ds(j * L, L))
                    buf[sl] = buf[sl] * v

    for i in range(NBUF):
        pltpu.async_copy(x_hbm.at[srcv.at[i]], rows[i], gsem[i])

    @pl.loop(0, CHUNKS // NBUF)
    def _(h):
        g0 = h * NBUF
        for i in range(NBUF):
            g = g0 + i
            pltpu.make_async_copy(x_hbm.at[srcv.at[g]], rows[i],
                                  gsem[i]).wait()
            _scale(rows[i], g)
            pltpu.async_copy(rows[i], acc.at[dstv.at[g]], ssem[i], add=True)
        for i in range(NBUF):
            g = g0 + i
            pltpu.make_async_copy(rows[i], acc.at[dstv.at[g]],
                                  ssem[i]).wait()

            @pl.when(g + NBUF < CHUNKS)
            def _():
                pltpu.async_copy(x_hbm.at[srcv.at[g + NBUF]], rows[i],
                                 gsem[i])

    plsc.subcore_barrier()

    pltpu.sync_copy(acc.at[pl.ds(s * RPW, RPW)],
                    out_hbm.at[pl.ds(c * N + s * RPW, RPW)])

    @pl.when(s == 0)
    def _():
        pltpu.sync_copy(acc.at[pl.ds(NS * RPW, TAIL)],
                        out_hbm.at[pl.ds(c * N + NS * RPW, TAIL)])


BN = 1000
NB = N // BN


def _mm_relu_body(p0_ref, p1_ref, w_ref, o_ref):
    wv = w_ref[...]
    y = lax.dot_general(p0_ref[...], wv[:D2], (((1,), (0,)), ((), ())),
                        preferred_element_type=jnp.float32,
                        precision=lax.Precision.HIGHEST)
    y += lax.dot_general(p1_ref[...], wv[D2:], (((1,), (0,)), ((), ())),
                         preferred_element_type=jnp.float32,
                         precision=lax.Precision.HIGHEST)
    h = jnp.maximum(y, 0.0)
    o_ref[0] = h[:, :D2]
    o_ref[1] = h[:, D2:]


def _tc_mm_relu(p, w):
    return pl.pallas_call(
        _mm_relu_body,
        grid=(NB,),
        in_specs=[
            pl.BlockSpec((BN, D2), lambda i: (i, 0)),
            pl.BlockSpec((BN, D2), lambda i: (i + NB, 0)),
            pl.BlockSpec((D, D), lambda i: (0, 0)),
        ],
        out_specs=pl.BlockSpec((NC, BN, D2), lambda i: (0, i, 0)),
        out_shape=jax.ShapeDtypeStruct((NC, N, D2), jnp.float32),
    )(p, p, w)


def _final_body(p0_ref, p1_ref, w_ref, c_ref, o_ref):
    wv = w_ref[...]
    y = lax.dot_general(p0_ref[...], wv[:D2], (((1,), (0,)), ((), ())),
                        preferred_element_type=jnp.float32,
                        precision=lax.Precision.HIGHEST)
    y += lax.dot_general(p1_ref[...], wv[D2:], (((1,), (0,)), ((), ())),
                         preferred_element_type=jnp.float32,
                         precision=lax.Precision.HIGHEST)
    h = jnp.maximum(y, 0.0)
    o_ref[...] = lax.dot_general(h, c_ref[...], (((1,), (0,)), ((), ())),
                                 preferred_element_type=jnp.float32,
                                 precision=lax.Precision.HIGHEST)


def _tc_final(p, w, cls):
    return pl.pallas_call(
        _final_body,
        grid=(NB,),
        in_specs=[
            pl.BlockSpec((BN, D2), lambda i: (i, 0)),
            pl.BlockSpec((BN, D2), lambda i: (i + NB, 0)),
            pl.BlockSpec((D, D), lambda i: (0, 0)),
            pl.BlockSpec((D, D), lambda i: (0, 0)),
        ],
        out_specs=pl.BlockSpec((BN, D), lambda i: (i, 0)),
        out_shape=jax.ShapeDtypeStruct((N, D), jnp.float32),
    )(p, p, w, cls)


def kernel(x, edge_index, edge_val, w0, w1, classifier):
    pad = ((0, 0), (0, EPP - EPS))
    src = jnp.pad(edge_index[0].reshape(NS, EPS), pad).reshape(NS, CHUNKS, K)
    dst = jnp.pad(edge_index[1].reshape(NS, EPS), pad).reshape(NS, CHUNKS, K)
    val = jnp.pad(edge_val.reshape(NS, EPS), pad).reshape(NS, CHUNKS, K)
    cls_pad = jnp.zeros((D, D), jnp.float32).at[:, :C].set(classifier)
    xcat = jnp.concatenate([x[:, :D2], x[:, D2:]], axis=0)

    p1 = _sc_aggregate(xcat, src, dst, val)
    h1 = _tc_mm_relu(p1, w0)
    p2 = _sc_aggregate(h1.reshape(NC * N, D2), src, dst, val)
    out = _tc_final(p2, w1, cls_pad)
    return out[:, :C]


# speedup vs baseline: 1.4145x; 1.4145x over previous
import functools

import jax
import jax.numpy as jnp
from jax import lax
from jax.experimental import pallas as pl
from jax.experimental.pallas import tpu as pltpu
from jax.experimental.pallas import tpu_sc as plsc

N = 10000
E = 320000
D = 128
C = 40

NC = 2
NS = 16
L = 16
D2 = D // NC
EPS = E // NS
K = 80
NBUF = 4
CHUNKS = 256
EPP = CHUNKS * K
RPW = 624
TAIL = N - NS * RPW
ZR = 104

_mesh = plsc.VectorSubcoreMesh(core_axis_name="c", subcore_axis_name="s")

_sc_params = pltpu.CompilerParams(
    needs_layout_passes=False, use_tc_tiling_on_sc=False)


@functools.partial(
    pl.kernel,
    out_type=jax.ShapeDtypeStruct((NC * N, D2), jnp.float32),
    mesh=_mesh,
    scratch_types=[
        pltpu.VMEM((CHUNKS, K), jnp.int32),
        pltpu.VMEM((CHUNKS, K), jnp.int32),
        pltpu.VMEM((CHUNKS, K), jnp.float32),
        [pltpu.VMEM((K, D2), jnp.float32)] * NBUF,
        pltpu.VMEM((ZR, D2), jnp.float32),
        pltpu.VMEM_SHARED((N, D2), jnp.float32),
        [pltpu.SemaphoreType.DMA] * NBUF,
        [pltpu.SemaphoreType.DMA] * NBUF,
    ],
    compiler_params=_sc_params,
)
def _sc_aggregate(x_hbm, src_hbm, dst_hbm, val_hbm, out_hbm,
                  srcv, dstv, valv, rows, zbuf, acc, gsem, ssem):
    c = lax.axis_index("c")
    s = lax.axis_index("s")

    zv = jnp.zeros((L,), jnp.float32)

    @pl.loop(0, ZR)
    def _(i):
        for j in range(D2 // L):
            zbuf[i, pl.ds(j * L, L)] = zv

    @pl.loop(0, RPW // ZR)
    def _(t):
        pltpu.sync_copy(zbuf, acc.at[pl.ds(s * RPW + t * ZR, ZR)])

    @pl.when(s == 0)
    def _():
        pltpu.sync_copy(zbuf.at[pl.ds(0, TAIL)], acc.at[pl.ds(NS * RPW, TAIL)])

    pltpu.sync_copy(src_hbm.at[s], srcv)
    pltpu.sync_copy(dst_hbm.at[s], dstv)
    pltpu.sync_copy(val_hbm.at[s], valv)

    coff = jnp.full((L,), c * N, jnp.int32)

    @pl.loop(0, CHUNKS)
    def _(g):
        for t in range(K // L):
            sl = (g, pl.ds(t * L, L))
            srcv[sl] = srcv[sl] + coff

    plsc.subcore_barrier()

    def _scale(buf, g):
        @pl.loop(0, K // L, unroll=K // L)
        def _(q):
            val16 = valv[g, pl.ds(q * L, L)]
            for l in range(L):
                v = jnp.full((L,), val16[l])
                e = q * L + l
                for j in range(D2 // L):
                    sl = (e, pl.ds(j * L, L))
                    buf[sl] = buf[sl] * v

    pltpu.async_copy(x_hbm.at[srcv.at[0]], rows[0], gsem[0])
    pltpu.async_copy(x_hbm.at[srcv.at[1]], rows[1], gsem[1])

    @pl.loop(0, CHUNKS // NBUF)
    def _(h):
        g0 = h * NBUF
        for i in range(NBUF):
            g = g0 + i
            x_buf = rows[i]
            y = (i + 2) % NBUF
            y_buf = rows[y]
            pltpu.make_async_copy(x_hbm.at[srcv.at[g]], x_buf,
                                  gsem[i]).wait()
            _scale(x_buf, g)
            pltpu.async_copy(x_buf, acc.at[dstv.at[g]], ssem[i], add=True)

            @pl.when(g >= 2)
            def _():
                pltpu.make_async_copy(y_buf, acc.at[dstv.at[g - 2]],
                                      ssem[y]).wait()

            @pl.when(g + 2 < CHUNKS)
            def _():
                pltpu.async_copy(x_hbm.at[srcv.at[g + 2]], y_buf, gsem[y])

    pltpu.make_async_copy(rows[(CHUNKS - 2) % NBUF],
                          acc.at[dstv.at[CHUNKS - 2]],
                          ssem[(CHUNKS - 2) % NBUF]).wait()
    pltpu.make_async_copy(rows[(CHUNKS - 1) % NBUF],
                          acc.at[dstv.at[CHUNKS - 1]],
                          ssem[(CHUNKS - 1) % NBUF]).wait()

    plsc.subcore_barrier()

    pltpu.sync_copy(acc.at[pl.ds(s * RPW, RPW)],
                    out_hbm.at[pl.ds(c * N + s * RPW, RPW)])

    @pl.when(s == 0)
    def _():
        pltpu.sync_copy(acc.at[pl.ds(NS * RPW, TAIL)],
                        out_hbm.at[pl.ds(c * N + NS * RPW, TAIL)])


BN = 1000
NB = N // BN


def _mm_relu_body(p0_ref, p1_ref, w_ref, o_ref):
    wv = w_ref[...]
    y = lax.dot_general(p0_ref[...], wv[:D2], (((1,), (0,)), ((), ())),
                        preferred_element_type=jnp.float32,
                        precision=lax.Precision.HIGHEST)
    y += lax.dot_general(p1_ref[...], wv[D2:], (((1,), (0,)), ((), ())),
                         preferred_element_type=jnp.float32,
                         precision=lax.Precision.HIGHEST)
    h = jnp.maximum(y, 0.0)
    o_ref[0] = h[:, :D2]
    o_ref[1] = h[:, D2:]


def _tc_mm_relu(p, w):
    return pl.pallas_call(
        _mm_relu_body,
        grid=(NB,),
        in_specs=[
            pl.BlockSpec((BN, D2), lambda i: (i, 0)),
            pl.BlockSpec((BN, D2), lambda i: (i + NB, 0)),
            pl.BlockSpec((D, D), lambda i: (0, 0)),
        ],
        out_specs=pl.BlockSpec((NC, BN, D2), lambda i: (0, i, 0)),
        out_shape=jax.ShapeDtypeStruct((NC, N, D2), jnp.float32),
    )(p, p, w)


def _final_body(p0_ref, p1_ref, w_ref, c_ref, o_ref):
    wv = w_ref[...]
    y = lax.dot_general(p0_ref[...], wv[:D2], (((1,), (0,)), ((), ())),
                        preferred_element_type=jnp.float32,
                        precision=lax.Precision.HIGHEST)
    y += lax.dot_general(p1_ref[...], wv[D2:], (((1,), (0,)), ((), ())),
                         preferred_element_type=jnp.float32,
                         precision=lax.Precision.HIGHEST)
    h = jnp.maximum(y, 0.0)
    o_ref[...] = lax.dot_general(h, c_ref[...], (((1,), (0,)), ((), ())),
                                 preferred_element_type=jnp.float32,
                                 precision=lax.Precision.HIGHEST)


def _tc_final(p, w, cls):
    return pl.pallas_call(
        _final_body,
        grid=(NB,),
        in_specs=[
            pl.BlockSpec((BN, D2), lambda i: (i, 0)),
            pl.BlockSpec((BN, D2), lambda i: (i + NB, 0)),
            pl.BlockSpec((D, D), lambda i: (0, 0)),
            pl.BlockSpec((D, D), lambda i: (0, 0)),
        ],
        out_specs=pl.BlockSpec((BN, D), lambda i: (i, 0)),
        out_shape=jax.ShapeDtypeStruct((N, D), jnp.float32),
    )(p, p, w, cls)


def kernel(x, edge_index, edge_val, w0, w1, classifier):
    pad = ((0, 0), (0, EPP - EPS))
    src = jnp.pad(edge_index[0].reshape(NS, EPS), pad).reshape(NS, CHUNKS, K)
    dst = jnp.pad(edge_index[1].reshape(NS, EPS), pad).reshape(NS, CHUNKS, K)
    val = jnp.pad(edge_val.reshape(NS, EPS), pad).reshape(NS, CHUNKS, K)
    cls_pad = jnp.zeros((D, D), jnp.float32).at[:, :C].set(classifier)
    xcat = jnp.concatenate([x[:, :D2], x[:, D2:]], axis=0)

    p1 = _sc_aggregate(xcat, src, dst, val)
    h1 = _tc_mm_relu(p1, w0)
    p2 = _sc_aggregate(h1.reshape(NC * N, D2), src, dst, val)
    out = _tc_final(p2, w1, cls_pad)
    return out[:, :C]
